# baseline (speedup 1.0000x reference)
import functools

import jax
import jax.numpy as jnp
from jax import lax
from jax.experimental import pallas as pl
from jax.experimental.pallas import tpu as pltpu
from jax.experimental.pallas import tpu_sc as plsc

N_ANC = 20000
G = 64
NT = 16
L = 16
NPAD = 20480
CHUNK = NPAD // NT
NGRP = CHUNK // L
EPS = 1.1920929e-07
LN2 = 0.6931471805599453
INT_MAX = 2147483647


def _vlog(x):
    bits = plsc.bitcast(x, jnp.int32)
    e = ((bits >> 23) & 255) - 127
    m = plsc.bitcast((bits & 0x007FFFFF) | 0x3F800000, jnp.float32)
    big = m > 1.4142135623730951
    m = jnp.where(big, m * 0.5, m)
    e = jnp.where(big, e + 1, e)
    s = (m - 1.0) / (m + 1.0)
    s2 = s * s
    p = 2.0 * s * (1.0 + s2 * (1.0 / 3.0 + s2 * (1.0 / 5.0 + s2 * (1.0 / 7.0))))
    return e.astype(jnp.float32) * LN2 + p


def _shuf(v, iota, k):
    return v.at[iota ^ k].get(mode="promise_in_bounds", unique_indices=True)


def _allmax(v, iota):
    for k in (1, 2, 4, 8):
        v = jnp.maximum(v, _shuf(v, iota, k))
    return v


def _allmin(v, iota):
    for k in (1, 2, 4, 8):
        v = jnp.minimum(v, _shuf(v, iota, k))
    return v


def _allsum(v, iota):
    for k in (1, 2, 4, 8):
        v = v + _shuf(v, iota, k)
    return v


def _smooth_l1(d):
    ad = jnp.abs(d)
    return jnp.where(ad < 1.0, 0.5 * ad * ad, ad - 0.5)


def _body(anc_hbm, pred_hbm, bb_hbm, out_hbm,
          anc_v, pred_v, bb_v, maxv_v, bestg_v, gmax_v, garg_v,
          allv_v, alli_v, part_v, res_v,
          sh_cv, sh_ci, sh_part):
    wid = lax.axis_index("s")
    base = wid * CHUNK
    iota = lax.iota(jnp.int32, L)
    zero = iota.astype(jnp.float32) * 0.0

    for c in range(4):
        pltpu.sync_copy(anc_hbm.at[pl.ds(c * NPAD + base, CHUNK)],
                        anc_v.at[pl.ds(c * CHUNK, CHUNK)])
        pltpu.sync_copy(pred_hbm.at[pl.ds(c * NPAD + base, CHUNK)],
                        pred_v.at[pl.ds(c * CHUNK, CHUNK)])
    pltpu.sync_copy(bb_hbm, bb_v.at[pl.ds(0, 4 * G * L)])
    def area_body(g, c):
        o = g * L
        gw = bb_v[pl.ds(2 * G * L + o, L)] - bb_v[pl.ds(0 * G * L + o, L)]
        gh = bb_v[pl.ds(3 * G * L + o, L)] - bb_v[pl.ds(1 * G * L + o, L)]
        bb_v[pl.ds(4 * G * L + o, L)] = gw * gh
        return c
    lax.fori_loop(0, G, area_body, 0)

    def init_g(g, c):
        gmax_v[pl.ds(g * L, L)] = zero - 1.0
        garg_v[pl.ds(g * L, L)] = iota * 0
        return c
    lax.fori_loop(0, G, init_g, 0)

    def grp_body(k, c):
        off = k * L
        aidx = base + off + iota
        ax1 = anc_v[pl.ds(off, L)]
        ay1 = anc_v[pl.ds(CHUNK + off, L)]
        ax2 = anc_v[pl.ds(2 * CHUNK + off, L)]
        ay2 = anc_v[pl.ds(3 * CHUNK + off, L)]
        area_a = (ax2 - ax1) * (ay2 - ay1)

        def g_body(g, carry):
            bv, bg = carry
            o = g * L
            gx1 = bb_v[pl.ds(o, L)]
            gy1 = bb_v[pl.ds(G * L + o, L)]
            gx2 = bb_v[pl.ds(2 * G * L + o, L)]
            gy2 = bb_v[pl.ds(3 * G * L + o, L)]
            garea = bb_v[pl.ds(4 * G * L + o, L)]
            tlx = jnp.maximum(ax1, gx1)
            tly = jnp.maximum(ay1, gy1)
            brx = jnp.minimum(ax2, gx2)
            bry = jnp.minimum(ay2, gy2)
            inter = (brx - tlx) * (bry - tly)
            valid = (tlx < brx) & (tly < bry)
            inter = jnp.where(valid, inter, 0.0)
            iou = inter / (area_a + garea - inter)
            upd = iou > bv
            bv = jnp.where(upd, iou, bv)
            bg = jnp.where(upd, g, bg)
            gm = gmax_v[pl.ds(g * L, L)]
            ga = garg_v[pl.ds(g * L, L)]
            u2 = iou > gm
            gmax_v[pl.ds(g * L, L)] = jnp.where(u2, iou, gm)
            garg_v[pl.ds(g * L, L)] = jnp.where(u2, aidx, ga)
            return bv, bg

        bv, bg = lax.fori_loop(0, G, g_body, (zero - 1.0, iota * 0))
        maxv_v[pl.ds(off, L)] = bv
        bestg_v[pl.ds(off, L)] = bg
        return c
    lax.fori_loop(0, NGRP, grp_body, 0)

    pltpu.sync_copy(gmax_v, sh_cv.at[pl.ds(wid * G * L, G * L)])
    pltpu.sync_copy(garg_v, sh_ci.at[pl.ds(wid * G * L, G * L)])
    plsc.subcore_barrier()
    pltpu.sync_copy(sh_cv, allv_v)
    pltpu.sync_copy(sh_ci, alli_v)

    def merge_g(g, c):
        cv0 = allv_v[pl.ds(g * L, L)]
        ci0 = alli_v[pl.ds(g * L, L)]

        def t_body(t, carry):
            cv, ci = carry
            v = allv_v[pl.ds(t * G * L + g * L, L)]
            i = alli_v[pl.ds(t * G * L + g * L, L)]
            better = (v > cv) | ((v == cv) & (i < ci))
            return jnp.where(better, v, cv), jnp.where(better, i, ci)

        cv, ci = lax.fori_loop(1, NT, t_body, (cv0, ci0))
        m = _allmax(cv, iota)
        cand = jnp.where(cv == m, ci, INT_MAX)
        garg_v[pl.ds(g * L, L)] = _allmin(cand, iota)
        return c
    lax.fori_loop(0, G, merge_g, 0)

    def p2_body(k, carry):
        lsum, csum = carry
        off = k * L
        aidx = base + off + iota
        bv = maxv_v[pl.ds(off, L)]
        bg = bestg_v[pl.ds(off, L)]

        def f_body(g, carry):
            bg, ff = carry
            a = garg_v[pl.ds(g * L, L)]
            match = aidx == a
            return jnp.where(match, g, bg), jnp.where(match, 1.0, ff)

        bg, forced_f = lax.fori_loop(0, G, f_body, (bg, zero))
        posf = jnp.where(bv >= 0.7, 1.0, forced_f)

        bgl = bg * L + iota
        gx1 = plsc.load_gather(bb_v, [bgl])
        gy1 = plsc.load_gather(bb_v, [bgl + G * L])
        gx2 = plsc.load_gather(bb_v, [bgl + 2 * G * L])
        gy2 = plsc.load_gather(bb_v, [bgl + 3 * G * L])

        ax1 = anc_v[pl.ds(off, L)]
        ay1 = anc_v[pl.ds(CHUNK + off, L)]
        ax2 = anc_v[pl.ds(2 * CHUNK + off, L)]
        ay2 = anc_v[pl.ds(3 * CHUNK + off, L)]
        w = ax2 - ax1
        h = ay2 - ay1
        cx = ax1 + 0.5 * w
        cy = ay1 + 0.5 * h
        w = jnp.maximum(w, EPS)
        h = jnp.maximum(h, EPS)
        bw = gx2 - gx1
        bh = gy2 - gy1
        bcx = gx1 + 0.5 * bw
        bcy = gy1 + 0.5 * bh
        dx = (bcx - cx) / w
        dy = (bcy - cy) / h
        dw = _vlog(bw / w)
        dh = _vlog(bh / h)
        px = pred_v[pl.ds(off, L)]
        py = pred_v[pl.ds(CHUNK + off, L)]
        pw = pred_v[pl.ds(2 * CHUNK + off, L)]
        ph = pred_v[pl.ds(3 * CHUNK + off, L)]
        l = (_smooth_l1(dx - px) + _smooth_l1(dy - py)
             + _smooth_l1(dw - pw) + _smooth_l1(dh - ph))
        return lsum + l * posf, csum + posf

    lsum, csum = lax.fori_loop(0, NGRP, p2_body, (zero, zero))

    part_v[pl.ds(0, L)] = lsum
    part_v[pl.ds(L, L)] = csum
    pltpu.sync_copy(part_v, sh_part.at[pl.ds(wid * 2 * L, 2 * L)])
    plsc.subcore_barrier()

    @pl.when(wid == 0)
    def _():
        pltpu.sync_copy(sh_part, allv_v.at[pl.ds(0, NT * 2 * L)])

        def s_body(t, carry):
            ls, cs = carry
            return (ls + allv_v[pl.ds(t * 2 * L, L)],
                    cs + allv_v[pl.ds(t * 2 * L + L, L)])

        ls, cs = lax.fori_loop(0, NT, s_body, (zero, zero))
        total = _allsum(ls, iota)
        cnt = _allsum(cs, iota)
        res_v[pl.ds(0, L)] = total / jnp.maximum(cnt, 1.0)
        pltpu.sync_copy(res_v, out_hbm)


_kcall = functools.partial(
    pl.kernel,
    out_type=jax.ShapeDtypeStruct((L,), jnp.float32),
    mesh=plsc.VectorSubcoreMesh(
        core_axis_name="c", subcore_axis_name="s", num_cores=1),
    compiler_params=pltpu.CompilerParams(needs_layout_passes=False),
    scratch_types=[
        pltpu.VMEM((4 * CHUNK,), jnp.float32),
        pltpu.VMEM((4 * CHUNK,), jnp.float32),
        pltpu.VMEM((5 * G * L,), jnp.float32),
        pltpu.VMEM((CHUNK,), jnp.float32),
        pltpu.VMEM((CHUNK,), jnp.int32),
        pltpu.VMEM((G * L,), jnp.float32),
        pltpu.VMEM((G * L,), jnp.int32),
        pltpu.VMEM((NT * G * L,), jnp.float32),
        pltpu.VMEM((NT * G * L,), jnp.int32),
        pltpu.VMEM((2 * L,), jnp.float32),
        pltpu.VMEM((L,), jnp.float32),
        pltpu.VMEM_SHARED((NT * G * L,), jnp.float32),
        pltpu.VMEM_SHARED((NT * G * L,), jnp.int32),
        pltpu.VMEM_SHARED((NT * 2 * L,), jnp.float32),
    ],
)(_body)


def kernel(pred_loc, anchors, bboxes, labels):
    del labels
    pad = NPAD - N_ANC
    anc = jnp.concatenate(
        [anchors.astype(jnp.float32),
         jnp.zeros((pad, 4), jnp.float32)]).T.reshape(-1)
    pred = jnp.concatenate(
        [pred_loc.astype(jnp.float32),
         jnp.zeros((pad, 4), jnp.float32)]).T.reshape(-1)
    bb = jnp.repeat(bboxes.astype(jnp.float32).T.reshape(-1), L)
    out = _kcall(anc, pred, bb)
    return out[0]


# speedup vs baseline: 1.8013x; 1.8013x over previous
import functools

import jax
import jax.numpy as jnp
from jax import lax
from jax.experimental import pallas as pl
from jax.experimental.pallas import tpu as pltpu
from jax.experimental.pallas import tpu_sc as plsc

N_ANC = 20000
G = 64
NT = 16
L = 16
NPAD = 20480
CHUNK = NPAD // NT
NGRP = CHUNK // L
EPS = 1.1920929e-07
LN2 = 0.6931471805599453
INT_MAX = 2147483647


def _vlog(x):
    bits = plsc.bitcast(x, jnp.int32)
    e = ((bits >> 23) & 255) - 127
    m = plsc.bitcast((bits & 0x007FFFFF) | 0x3F800000, jnp.float32)
    big = m > 1.4142135623730951
    m = jnp.where(big, m * 0.5, m)
    e = jnp.where(big, e + 1, e)
    s = (m - 1.0) / (m + 1.0)
    s2 = s * s
    p = 2.0 * s * (1.0 + s2 * (1.0 / 3.0 + s2 * (1.0 / 5.0 + s2 * (1.0 / 7.0))))
    return e.astype(jnp.float32) * LN2 + p


def _shuf(v, iota, k):
    return v.at[iota ^ k].get(mode="promise_in_bounds", unique_indices=True)


def _allmax(v, iota):
    for k in (1, 2, 4, 8):
        v = jnp.maximum(v, _shuf(v, iota, k))
    return v


def _allmin(v, iota):
    for k in (1, 2, 4, 8):
        v = jnp.minimum(v, _shuf(v, iota, k))
    return v


def _allsum(v, iota):
    for k in (1, 2, 4, 8):
        v = v + _shuf(v, iota, k)
    return v


def _smooth_l1(d):
    ad = jnp.abs(d)
    return jnp.where(ad < 1.0, 0.5 * ad * ad, ad - 0.5)


def _body(anc_hbm, pred_hbm, bb_hbm, out_hbm,
          anc_v, pred_v, bb_v, maxv_v, bestg_v, forced_v, gmax_v, garg_v,
          allv_v, alli_v, part_v, res_v,
          sh_cv, sh_ci, sh_part):
    wid = lax.axis_index("s")
    base = wid * CHUNK
    iota = lax.iota(jnp.int32, L)
    zero = iota.astype(jnp.float32) * 0.0

    for c in range(4):
        pltpu.sync_copy(anc_hbm.at[pl.ds(c * NPAD + base, CHUNK)],
                        anc_v.at[pl.ds(c * CHUNK, CHUNK)])
        pltpu.sync_copy(pred_hbm.at[pl.ds(c * NPAD + base, CHUNK)],
                        pred_v.at[pl.ds(c * CHUNK, CHUNK)])
    pltpu.sync_copy(bb_hbm, bb_v.at[pl.ds(0, 4 * G * L)])
    def area_body(g, c):
        o = g * L
        gw = bb_v[pl.ds(2 * G * L + o, L)] - bb_v[pl.ds(0 * G * L + o, L)]
        gh = bb_v[pl.ds(3 * G * L + o, L)] - bb_v[pl.ds(1 * G * L + o, L)]
        bb_v[pl.ds(4 * G * L + o, L)] = gw * gh
        return c
    lax.fori_loop(0, G, area_body, 0)

    def init_g(g, c):
        gmax_v[pl.ds(g * L, L)] = zero - 1.0
        garg_v[pl.ds(g * L, L)] = iota * 0
        return c
    lax.fori_loop(0, G, init_g, 0)

    UB = 2

    def grp_body(kb, c):
        offs = [kb * (UB * L) + u * L for u in range(UB)]
        aidx = [base + o + iota for o in offs]
        ax1 = [anc_v[pl.ds(o, L)] for o in offs]
        ay1 = [anc_v[pl.ds(CHUNK + o, L)] for o in offs]
        ax2 = [anc_v[pl.ds(2 * CHUNK + o, L)] for o in offs]
        ay2 = [anc_v[pl.ds(3 * CHUNK + o, L)] for o in offs]
        area_a = [(ax2[u] - ax1[u]) * (ay2[u] - ay1[u]) for u in range(UB)]

        def g_body(g, carry):
            bv = list(carry[:UB])
            bg = list(carry[UB:])
            o = g * L
            gx1 = bb_v[pl.ds(o, L)]
            gy1 = bb_v[pl.ds(G * L + o, L)]
            gx2 = bb_v[pl.ds(2 * G * L + o, L)]
            gy2 = bb_v[pl.ds(3 * G * L + o, L)]
            garea = bb_v[pl.ds(4 * G * L + o, L)]
            gm = gmax_v[pl.ds(o, L)]
            ga = garg_v[pl.ds(o, L)]
            for u in range(UB):
                w = jnp.minimum(ax2[u], gx2) - jnp.maximum(ax1[u], gx1)
                h = jnp.minimum(ay2[u], gy2) - jnp.maximum(ay1[u], gy1)
                inter = jnp.maximum(w, 0.0) * jnp.maximum(h, 0.0)
                iou = inter / (area_a[u] + garea - inter)
                upd = iou > bv[u]
                bv[u] = jnp.where(upd, iou, bv[u])
                bg[u] = jnp.where(upd, g, bg[u])
                u2 = iou > gm
                gm = jnp.where(u2, iou, gm)
                ga = jnp.where(u2, aidx[u], ga)
            gmax_v[pl.ds(o, L)] = gm
            garg_v[pl.ds(o, L)] = ga
            return tuple(bv) + tuple(bg)

        init = tuple(zero - 1.0 for _ in range(UB)) + tuple(
            iota * 0 for _ in range(UB))
        res = lax.fori_loop(0, G, g_body, init, unroll=2)
        for u in range(UB):
            maxv_v[pl.ds(offs[u], L)] = res[u]
            bestg_v[pl.ds(offs[u], L)] = res[UB + u]
        return c
    lax.fori_loop(0, NGRP // UB, grp_body, 0)

    pltpu.sync_copy(gmax_v, sh_cv.at[pl.ds(wid * G * L, G * L)])
    pltpu.sync_copy(garg_v, sh_ci.at[pl.ds(wid * G * L, G * L)])
    plsc.subcore_barrier()
    pltpu.sync_copy(sh_cv, allv_v)
    pltpu.sync_copy(sh_ci, alli_v)

    def merge_g(g, c):
        cv0 = allv_v[pl.ds(g * L, L)]
        ci0 = alli_v[pl.ds(g * L, L)]

        def t_body(t, carry):
            cv, ci = carry
            v = allv_v[pl.ds(t * G * L + g * L, L)]
            i = alli_v[pl.ds(t * G * L + g * L, L)]
            better = (v > cv) | ((v == cv) & (i < ci))
            return jnp.where(better, v, cv), jnp.where(better, i, ci)

        cv, ci = lax.fori_loop(1, NT, t_body, (cv0, ci0), unroll=5)
        m = _allmax(cv, iota)
        cand = jnp.where(cv == m, ci, INT_MAX)
        garg_v[pl.ds(g * L, L)] = _allmin(cand, iota)
        return c
    lax.fori_loop(0, G, merge_g, 0)

    def z_body(k, c):
        forced_v[pl.ds(k * L, L)] = zero
        return c
    lax.fori_loop(0, NGRP, z_body, 0)

    lane0 = iota == 0
    one_v = zero + 1.0

    def ov_body(g, c):
        a = garg_v[pl.ds(g * L, L)]
        loc = a - base
        hit = lane0 & (loc >= 0) & (loc < CHUNK)
        plsc.store_scatter(bestg_v, [loc], iota * 0 + g, mask=hit)
        plsc.store_scatter(forced_v, [loc], one_v, mask=hit)
        return c
    lax.fori_loop(0, G, ov_body, 0)

    def p2_body(k, carry):
        lsum, csum = carry
        off = k * L
        bv = maxv_v[pl.ds(off, L)]
        bg = bestg_v[pl.ds(off, L)]
        forced_f = forced_v[pl.ds(off, L)]
        posf = jnp.where(bv >= 0.7, 1.0, forced_f)

        bgl = bg * L + iota
        gx1 = plsc.load_gather(bb_v, [bgl])
        gy1 = plsc.load_gather(bb_v, [bgl + G * L])
        gx2 = plsc.load_gather(bb_v, [bgl + 2 * G * L])
        gy2 = plsc.load_gather(bb_v, [bgl + 3 * G * L])

        ax1 = anc_v[pl.ds(off, L)]
        ay1 = anc_v[pl.ds(CHUNK + off, L)]
        ax2 = anc_v[pl.ds(2 * CHUNK + off, L)]
        ay2 = anc_v[pl.ds(3 * CHUNK + off, L)]
        w = ax2 - ax1
        h = ay2 - ay1
        cx = ax1 + 0.5 * w
        cy = ay1 + 0.5 * h
        w = jnp.maximum(w, EPS)
        h = jnp.maximum(h, EPS)
        bw = gx2 - gx1
        bh = gy2 - gy1
        bcx = gx1 + 0.5 * bw
        bcy = gy1 + 0.5 * bh
        dx = (bcx - cx) / w
        dy = (bcy - cy) / h
        dw = _vlog(bw / w)
        dh = _vlog(bh / h)
        px = pred_v[pl.ds(off, L)]
        py = pred_v[pl.ds(CHUNK + off, L)]
        pw = pred_v[pl.ds(2 * CHUNK + off, L)]
        ph = pred_v[pl.ds(3 * CHUNK + off, L)]
        l = (_smooth_l1(dx - px) + _smooth_l1(dy - py)
             + _smooth_l1(dw - pw) + _smooth_l1(dh - ph))
        return lsum + l * posf, csum + posf

    lsum, csum = lax.fori_loop(0, NGRP, p2_body, (zero, zero), unroll=2)

    part_v[pl.ds(0, L)] = lsum
    part_v[pl.ds(L, L)] = csum
    pltpu.sync_copy(part_v, sh_part.at[pl.ds(wid * 2 * L, 2 * L)])
    plsc.subcore_barrier()

    @pl.when(wid == 0)
    def _():
        pltpu.sync_copy(sh_part, allv_v.at[pl.ds(0, NT * 2 * L)])

        def s_body(t, carry):
            ls, cs = carry
            return (ls + allv_v[pl.ds(t * 2 * L, L)],
                    cs + allv_v[pl.ds(t * 2 * L + L, L)])

        ls, cs = lax.fori_loop(0, NT, s_body, (zero, zero))
        total = _allsum(ls, iota)
        cnt = _allsum(cs, iota)
        res_v[pl.ds(0, L)] = total / jnp.maximum(cnt, 1.0)
        pltpu.sync_copy(res_v, out_hbm)


_kcall = functools.partial(
    pl.kernel,
    out_type=jax.ShapeDtypeStruct((L,), jnp.float32),
    mesh=plsc.VectorSubcoreMesh(
        core_axis_name="c", subcore_axis_name="s", num_cores=1),
    compiler_params=pltpu.CompilerParams(needs_layout_passes=False),
    scratch_types=[
        pltpu.VMEM((4 * CHUNK,), jnp.float32),
        pltpu.VMEM((4 * CHUNK,), jnp.float32),
        pltpu.VMEM((5 * G * L,), jnp.float32),
        pltpu.VMEM((CHUNK,), jnp.float32),
        pltpu.VMEM((CHUNK,), jnp.int32),
        pltpu.VMEM((CHUNK,), jnp.float32),
        pltpu.VMEM((G * L,), jnp.float32),
        pltpu.VMEM((G * L,), jnp.int32),
        pltpu.VMEM((NT * G * L,), jnp.float32),
        pltpu.VMEM((NT * G * L,), jnp.int32),
        pltpu.VMEM((2 * L,), jnp.float32),
        pltpu.VMEM((L,), jnp.float32),
        pltpu.VMEM_SHARED((NT * G * L,), jnp.float32),
        pltpu.VMEM_SHARED((NT * G * L,), jnp.int32),
        pltpu.VMEM_SHARED((NT * 2 * L,), jnp.float32),
    ],
)(_body)


def kernel(pred_loc, anchors, bboxes, labels):
    del labels
    pad = NPAD - N_ANC
    anc = jnp.concatenate(
        [anchors.astype(jnp.float32),
         jnp.zeros((pad, 4), jnp.float32)]).T.reshape(-1)
    pred = jnp.concatenate(
        [pred_loc.astype(jnp.float32),
         jnp.zeros((pad, 4), jnp.float32)]).T.reshape(-1)
    bb = jnp.repeat(bboxes.astype(jnp.float32).T.reshape(-1), L)
    out = _kcall(anc, pred, bb)
    return out[0]


# speedup vs baseline: 2.3610x; 1.3107x over previous
import functools

import jax
import jax.numpy as jnp
from jax import lax
from jax.experimental import pallas as pl
from jax.experimental.pallas import tpu as pltpu
from jax.experimental.pallas import tpu_sc as plsc

N_ANC = 20000
G = 64
NT = 16
L = 16
NPAD = 20480
CHUNK = NPAD // NT
NGRP = CHUNK // L
EPS = 1.1920929e-07
LN2 = 0.6931471805599453
INT_MAX = 2147483647


def _vlog(x):
    bits = plsc.bitcast(x, jnp.int32)
    e = ((bits >> 23) & 255) - 127
    m = plsc.bitcast((bits & 0x007FFFFF) | 0x3F800000, jnp.float32)
    big = m > 1.4142135623730951
    m = jnp.where(big, m * 0.5, m)
    e = jnp.where(big, e + 1, e)
    s = (m - 1.0) / (m + 1.0)
    s2 = s * s
    p = 2.0 * s * (1.0 + s2 * (1.0 / 3.0 + s2 * (1.0 / 5.0 + s2 * (1.0 / 7.0))))
    return e.astype(jnp.float32) * LN2 + p


def _shuf(v, iota, k):
    return v.at[iota ^ k].get(mode="promise_in_bounds", unique_indices=True)


def _allmax(v, iota):
    for k in (1, 2, 4, 8):
        v = jnp.maximum(v, _shuf(v, iota, k))
    return v


def _allmin(v, iota):
    for k in (1, 2, 4, 8):
        v = jnp.minimum(v, _shuf(v, iota, k))
    return v


def _allsum(v, iota):
    for k in (1, 2, 4, 8):
        v = v + _shuf(v, iota, k)
    return v


def _smooth_l1(d):
    ad = jnp.abs(d)
    return jnp.where(ad < 1.0, 0.5 * ad * ad, ad - 0.5)


def _body(anc_hbm, pred_hbm, bb_hbm, out_hbm,
          anc_v, pred_v, bb_v, maxv_v, bestg_v, forced_v, gmax_v, garg_v,
          allv_v, alli_v, part_v, res_v,
          sh_cv, sh_ci, sh_part):
    wid = lax.axis_index("s")
    base = wid * CHUNK
    iota = lax.iota(jnp.int32, L)
    zero = iota.astype(jnp.float32) * 0.0

    for c in range(4):
        pltpu.sync_copy(anc_hbm.at[pl.ds(c * NPAD + base, CHUNK)],
                        anc_v.at[pl.ds(c * CHUNK, CHUNK)])
        pltpu.sync_copy(pred_hbm.at[pl.ds(c * NPAD + base, CHUNK)],
                        pred_v.at[pl.ds(c * CHUNK, CHUNK)])
    pltpu.sync_copy(bb_hbm, bb_v.at[pl.ds(0, 4 * G * L)])
    def area_body(g, c):
        o = g * L
        gw = bb_v[pl.ds(2 * G * L + o, L)] - bb_v[pl.ds(0 * G * L + o, L)]
        gh = bb_v[pl.ds(3 * G * L + o, L)] - bb_v[pl.ds(1 * G * L + o, L)]
        bb_v[pl.ds(4 * G * L + o, L)] = gw * gh
        return c
    lax.fori_loop(0, G, area_body, 0)

    def init_g(g, c):
        gmax_v[pl.ds(g * L, L)] = zero - 1.0
        garg_v[pl.ds(g * L, L)] = iota * 0
        return c
    lax.fori_loop(0, G, init_g, 0)

    UB = 4

    def grp_body(kb, c):
        offs = [kb * (UB * L) + u * L for u in range(UB)]
        aidx = [base + o + iota for o in offs]
        ax1 = [anc_v[pl.ds(o, L)] for o in offs]
        ay1 = [anc_v[pl.ds(CHUNK + o, L)] for o in offs]
        ax2 = [anc_v[pl.ds(2 * CHUNK + o, L)] for o in offs]
        ay2 = [anc_v[pl.ds(3 * CHUNK + o, L)] for o in offs]
        area_a = [(ax2[u] - ax1[u]) * (ay2[u] - ay1[u]) for u in range(UB)]

        def g_body(g, carry):
            bv = list(carry[:UB])
            bg = list(carry[UB:])
            o = g * L
            gx1 = bb_v[pl.ds(o, L)]
            gy1 = bb_v[pl.ds(G * L + o, L)]
            gx2 = bb_v[pl.ds(2 * G * L + o, L)]
            gy2 = bb_v[pl.ds(3 * G * L + o, L)]
            garea = bb_v[pl.ds(4 * G * L + o, L)]
            gm = gmax_v[pl.ds(o, L)]
            ga = garg_v[pl.ds(o, L)]
            for u in range(UB):
                w = jnp.minimum(ax2[u], gx2) - jnp.maximum(ax1[u], gx1)
                h = jnp.minimum(ay2[u], gy2) - jnp.maximum(ay1[u], gy1)
                inter = jnp.maximum(w, 0.0) * jnp.maximum(h, 0.0)
                iou = inter / (area_a[u] + garea - inter)
                upd = iou > bv[u]
                bv[u] = jnp.where(upd, iou, bv[u])
                bg[u] = jnp.where(upd, g, bg[u])
                u2 = iou > gm
                gm = jnp.where(u2, iou, gm)
                ga = jnp.where(u2, aidx[u], ga)
            gmax_v[pl.ds(o, L)] = gm
            garg_v[pl.ds(o, L)] = ga
            return tuple(bv) + tuple(bg)

        init = tuple(zero - 1.0 for _ in range(UB)) + tuple(
            iota * 0 for _ in range(UB))
        res = lax.fori_loop(0, G, g_body, init, unroll=2)
        for u in range(UB):
            maxv_v[pl.ds(offs[u], L)] = res[u]
            bestg_v[pl.ds(offs[u], L)] = res[UB + u]
        return c
    lax.fori_loop(0, NGRP // UB, grp_body, 0)

    pltpu.sync_copy(gmax_v, sh_cv.at[pl.ds(wid * G * L, G * L)])
    pltpu.sync_copy(garg_v, sh_ci.at[pl.ds(wid * G * L, G * L)])
    plsc.subcore_barrier()
    pltpu.sync_copy(sh_cv, allv_v)
    pltpu.sync_copy(sh_ci, alli_v)

    def merge_g(g, c):
        cv0 = allv_v[pl.ds(g * L, L)]
        ci0 = alli_v[pl.ds(g * L, L)]

        def t_body(t, carry):
            cv, ci = carry
            v = allv_v[pl.ds(t * G * L + g * L, L)]
            i = alli_v[pl.ds(t * G * L + g * L, L)]
            better = (v > cv) | ((v == cv) & (i < ci))
            return jnp.where(better, v, cv), jnp.where(better, i, ci)

        cv, ci = lax.fori_loop(1, NT, t_body, (cv0, ci0), unroll=5)
        m = _allmax(cv, iota)
        cand = jnp.where(cv == m, ci, INT_MAX)
        garg_v[pl.ds(g * L, L)] = _allmin(cand, iota)
        return c
    lax.fori_loop(0, G, merge_g, 0)

    def z_body(k, c):
        forced_v[pl.ds(k * L, L)] = zero
        return c
    lax.fori_loop(0, NGRP, z_body, 0)

    lane0 = iota == 0
    one_v = zero + 1.0

    def ov_body(g, c):
        a = garg_v[pl.ds(g * L, L)]
        loc = a - base
        hit = lane0 & (loc >= 0) & (loc < CHUNK)
        plsc.store_scatter(bestg_v, [loc], iota * 0 + g, mask=hit)
        plsc.store_scatter(forced_v, [loc], one_v, mask=hit)
        return c
    lax.fori_loop(0, G, ov_body, 0)

    def p2_body(k, carry):
        lsum, csum = carry
        off = k * L
        bv = maxv_v[pl.ds(off, L)]
        bg = bestg_v[pl.ds(off, L)]
        forced_f = forced_v[pl.ds(off, L)]
        posf = jnp.where(bv >= 0.7, 1.0, forced_f)

        bgl = bg * L + iota
        gx1 = plsc.load_gather(bb_v, [bgl])
        gy1 = plsc.load_gather(bb_v, [bgl + G * L])
        gx2 = plsc.load_gather(bb_v, [bgl + 2 * G * L])
        gy2 = plsc.load_gather(bb_v, [bgl + 3 * G * L])

        ax1 = anc_v[pl.ds(off, L)]
        ay1 = anc_v[pl.ds(CHUNK + off, L)]
        ax2 = anc_v[pl.ds(2 * CHUNK + off, L)]
        ay2 = anc_v[pl.ds(3 * CHUNK + off, L)]
        w = ax2 - ax1
        h = ay2 - ay1
        cx = ax1 + 0.5 * w
        cy = ay1 + 0.5 * h
        w = jnp.maximum(w, EPS)
        h = jnp.maximum(h, EPS)
        bw = gx2 - gx1
        bh = gy2 - gy1
        bcx = gx1 + 0.5 * bw
        bcy = gy1 + 0.5 * bh
        dx = (bcx - cx) / w
        dy = (bcy - cy) / h
        dw = _vlog(bw / w)
        dh = _vlog(bh / h)
        px = pred_v[pl.ds(off, L)]
        py = pred_v[pl.ds(CHUNK + off, L)]
        pw = pred_v[pl.ds(2 * CHUNK + off, L)]
        ph = pred_v[pl.ds(3 * CHUNK + off, L)]
        l = (_smooth_l1(dx - px) + _smooth_l1(dy - py)
             + _smooth_l1(dw - pw) + _smooth_l1(dh - ph))
        return lsum + l * posf, csum + posf

    lsum, csum = lax.fori_loop(0, NGRP, p2_body, (zero, zero), unroll=2)

    part_v[pl.ds(0, L)] = lsum
    part_v[pl.ds(L, L)] = csum
    pltpu.sync_copy(part_v, sh_part.at[pl.ds(wid * 2 * L, 2 * L)])
    plsc.subcore_barrier()

    @pl.when(wid == 0)
    def _():
        pltpu.sync_copy(sh_part, allv_v.at[pl.ds(0, NT * 2 * L)])

        def s_body(t, carry):
            ls, cs = carry
            return (ls + allv_v[pl.ds(t * 2 * L, L)],
                    cs + allv_v[pl.ds(t * 2 * L + L, L)])

        ls, cs = lax.fori_loop(0, NT, s_body, (zero, zero))
        total = _allsum(ls, iota)
        cnt = _allsum(cs, iota)
        res_v[pl.ds(0, L)] = total / jnp.maximum(cnt, 1.0)
        pltpu.sync_copy(res_v, out_hbm)


_kcall = functools.partial(
    pl.kernel,
    out_type=jax.ShapeDtypeStruct((L,), jnp.float32),
    mesh=plsc.VectorSubcoreMesh(
        core_axis_name="c", subcore_axis_name="s", num_cores=1),
    compiler_params=pltpu.CompilerParams(needs_layout_passes=False),
    scratch_types=[
        pltpu.VMEM((4 * CHUNK,), jnp.float32),
        pltpu.VMEM((4 * CHUNK,), jnp.float32),
        pltpu.VMEM((5 * G * L,), jnp.float32),
        pltpu.VMEM((CHUNK,), jnp.float32),
        pltpu.VMEM((CHUNK,), jnp.int32),
        pltpu.VMEM((CHUNK,), jnp.float32),
        pltpu.VMEM((G * L,), jnp.float32),
        pltpu.VMEM((G * L,), jnp.int32),
        pltpu.VMEM((NT * G * L,), jnp.float32),
        pltpu.VMEM((NT * G * L,), jnp.int32),
        pltpu.VMEM((2 * L,), jnp.float32),
        pltpu.VMEM((L,), jnp.float32),
        pltpu.VMEM_SHARED((NT * G * L,), jnp.float32),
        pltpu.VMEM_SHARED((NT * G * L,), jnp.int32),
        pltpu.VMEM_SHARED((NT * 2 * L,), jnp.float32),
    ],
)(_body)


def kernel(pred_loc, anchors, bboxes, labels):
    del labels
    pad = NPAD - N_ANC
    anc = jnp.concatenate(
        [anchors.astype(jnp.float32),
         jnp.zeros((pad, 4), jnp.float32)]).T.reshape(-1)
    pred = jnp.concatenate(
        [pred_loc.astype(jnp.float32),
         jnp.zeros((pad, 4), jnp.float32)]).T.reshape(-1)
    bb = jnp.repeat(bboxes.astype(jnp.float32).T.reshape(-1), L)
    out = _kcall(anc, pred, bb)
    return out[0]
